# Initial kernel scaffold; baseline (speedup 1.0000x reference)
import jax
import jax.numpy as jnp
from jax.experimental import pallas as pl


def kernel(h, u, pos_state, pos_action, a2s_edge_index, a2s_dis, s2s_edge_index, s2s_dis, a2s_W1, a2s_b1, a2s_W2, a2s_b2, s2s_W1, s2s_b1, s2s_W2, s2s_b2, upd_W1, upd_b1, upd_W2, upd_b2):
    raise NotImplementedError("write your pallas kernel here")


# speedup vs baseline: 2.8127x; 2.8127x over previous
import functools

import jax
import jax.numpy as jnp
from jax import lax
from jax.experimental import pallas as pl
from jax.experimental.pallas import tpu as pltpu
from jax.experimental.pallas import tpu_sc as plsc

N_S = 10000
N_A = 10000
E_A2S = 320000
E_S2S = 320000
H_DIM = 128
MLP_H = 32

NC = 2
NSUB = 16
NW = NC * NSUB
EPW = E_A2S // NW
CH = 80
NCHUNK = EPW // CH

ROWS_PT = 624
TAIL_BASE = ROWS_PT * NSUB
TAIL_ROWS = N_S - TAIL_BASE

_MESH = dict(core_axis_name="c", subcore_axis_name="s", num_cores=NC,
             num_subcores=NSUB)


def _leaky(x):
    return jnp.where(x >= 0, x, 0.01 * x)


def _proj_body(pos_action, u, pos_state, h,
               wa_pos, wa_u, ba1, wa_q,
               ws_pos, ws_h, bs1, ws_q,
               pa_o, qa_o, ps_o, qs_o):
    f32 = jnp.float32
    pa_o[...] = (jnp.dot(pos_action[...], wa_pos[...], preferred_element_type=f32)
                 + jnp.dot(u[...], wa_u[...], preferred_element_type=f32)
                 + ba1[...][None, :])
    qa_o[...] = jnp.dot(pos_state[...], wa_q[...], preferred_element_type=f32)
    ps_o[...] = (jnp.dot(pos_state[...], ws_pos[...], preferred_element_type=f32)
                 + jnp.dot(h[...], ws_h[...], preferred_element_type=f32)
                 + bs1[...][None, :])
    qs_o[...] = jnp.dot(pos_state[...], ws_q[...], preferred_element_type=f32)


def _projections(pos_action, u, pos_state, h, a2s_W1, a2s_b1, s2s_W1, s2s_b1):
    f32 = jnp.float32
    out_shape = [
        jax.ShapeDtypeStruct((N_A, MLP_H), f32),
        jax.ShapeDtypeStruct((N_S, MLP_H), f32),
        jax.ShapeDtypeStruct((N_S, MLP_H), f32),
        jax.ShapeDtypeStruct((N_S, MLP_H), f32),
    ]
    return pl.pallas_call(_proj_body, out_shape=out_shape)(
        pos_action, u, pos_state, h,
        a2s_W1[0:2], a2s_W1[5:], a2s_b1, a2s_W1[2:4],
        s2s_W1[0:2], s2s_W1[5:], s2s_b1, s2s_W1[2:4])


def _gather_body(pa, qa, ps, qs, asrc, adst, ssrc, sdst,
                 oa1, oa2, os1, os2, idx_v, rows_v, sem):
    wid = lax.axis_index("s") * NC + lax.axis_index("c")
    base0 = wid * EPW
    for tab, src, out in ((pa, asrc, oa1), (qa, adst, oa2),
                          (ps, ssrc, os1), (qs, sdst, os2)):
        def chunk(i, _, tab=tab, src=src, out=out):
            base = base0 + i * CH
            pltpu.sync_copy(src.at[pl.ds(base, CH)], idx_v)
            pltpu.async_copy(tab.at[idx_v], rows_v, sem).wait()
            pltpu.sync_copy(rows_v, out.at[pl.ds(base, CH)])
            return _
        lax.fori_loop(0, NCHUNK, chunk, 0, unroll=False)


def _gather(pa, qa, ps, qs, asrc, adst, ssrc, sdst):
    f32 = jnp.float32
    out_type = [
        jax.ShapeDtypeStruct((E_A2S, MLP_H), f32),
        jax.ShapeDtypeStruct((E_A2S, MLP_H), f32),
        jax.ShapeDtypeStruct((E_S2S, MLP_H), f32),
        jax.ShapeDtypeStruct((E_S2S, MLP_H), f32),
    ]
    fn = pl.kernel(
        _gather_body,
        out_type=out_type,
        mesh=plsc.VectorSubcoreMesh(**_MESH),
        compiler_params=pltpu.CompilerParams(use_tc_tiling_on_sc=False),
        scratch_types=[
            pltpu.VMEM((CH,), jnp.int32),
            pltpu.VMEM((CH, MLP_H), f32),
            pltpu.SemaphoreType.DMA,
        ],
    )
    return fn(pa, qa, ps, qs, asrc, adst, ssrc, sdst)


_BE = 4000


def _msg_body(g1, g2, dis, wdis, w2, b2, out):
    f32 = jnp.float32
    pre = g1[...] + g2[...] + jnp.dot(dis[...], wdis[...],
                                      preferred_element_type=f32)
    z = _leaky(pre)
    out[...] = jnp.tanh(jnp.dot(z, w2[...], preferred_element_type=f32)
                        + b2[...][None, :])


def _messages(g1, g2, dis, wdis, w2, b2, n_edges):
    f32 = jnp.float32
    grid = n_edges // _BE
    return pl.pallas_call(
        _msg_body,
        out_shape=jax.ShapeDtypeStruct((n_edges, H_DIM), f32),
        grid=(grid,),
        in_specs=[
            pl.BlockSpec((_BE, MLP_H), lambda i: (i, 0)),
            pl.BlockSpec((_BE, MLP_H), lambda i: (i, 0)),
            pl.BlockSpec((_BE, 1), lambda i: (i, 0)),
            pl.BlockSpec((1, MLP_H), lambda i: (0, 0)),
            pl.BlockSpec((MLP_H, H_DIM), lambda i: (0, 0)),
            pl.BlockSpec((H_DIM,), lambda i: (0,)),
        ],
        out_specs=pl.BlockSpec((_BE, H_DIM), lambda i: (i, 0)),
    )(g1, g2, dis, wdis, w2, b2)


def _scatter_body(msga, adst, msgs, sdst, zeros,
                  outa, outs, idx_v, rows_v, accum, sem):
    cid = lax.axis_index("c")
    sid = lax.axis_index("s")
    wid = sid * NC + cid
    base0 = wid * EPW
    rbase = sid * ROWS_PT
    last = sid == NSUB - 1
    for msg, dst, out in ((msga, adst, outa), (msgs, sdst, outs)):
        pltpu.sync_copy(zeros.at[pl.ds(rbase, ROWS_PT)],
                        accum.at[pl.ds(rbase, ROWS_PT)])

        @pl.when(last)
        def _zero_tail():
            pltpu.sync_copy(zeros.at[pl.ds(TAIL_BASE, TAIL_ROWS)],
                            accum.at[pl.ds(TAIL_BASE, TAIL_ROWS)])
        plsc.subcore_barrier()

        def chunk(i, _, msg=msg, dst=dst):
            base = base0 + i * CH
            pltpu.sync_copy(dst.at[pl.ds(base, CH)], idx_v)
            pltpu.sync_copy(msg.at[pl.ds(base, CH)], rows_v)
            pltpu.sync_copy(rows_v, accum.at[idx_v], add=True)
            return _
        lax.fori_loop(0, NCHUNK, chunk, 0, unroll=False)
        plsc.subcore_barrier()
        pltpu.sync_copy(accum.at[pl.ds(rbase, ROWS_PT)],
                        out.at[cid, pl.ds(rbase, ROWS_PT)])

        @pl.when(last)
        def _dump_tail():
            pltpu.sync_copy(accum.at[pl.ds(TAIL_BASE, TAIL_ROWS)],
                            out.at[cid, pl.ds(TAIL_BASE, TAIL_ROWS)])
        plsc.subcore_barrier()


def _segment_sums(msga, adst, msgs, sdst, zeros):
    f32 = jnp.float32
    out_type = [
        jax.ShapeDtypeStruct((NC, N_S, H_DIM), f32),
        jax.ShapeDtypeStruct((NC, N_S, H_DIM), f32),
    ]
    fn = pl.kernel(
        _scatter_body,
        out_type=out_type,
        mesh=plsc.VectorSubcoreMesh(**_MESH),
        scratch_types=[
            pltpu.VMEM((CH,), jnp.int32),
            pltpu.VMEM((CH, H_DIM), f32),
            pltpu.VMEM_SHARED((N_S, H_DIM), f32),
            pltpu.SemaphoreType.DMA,
        ],
    )
    return fn(msga, adst, msgs, sdst, zeros)


_BN = 2000


def _upd_body(pos_state, h, sua, sus, wp, wh, wu, wsh, b1, w2, b2, out):
    f32 = jnp.float32
    pre = (jnp.dot(pos_state[...], wp[...], preferred_element_type=f32)
           + jnp.dot(h[...], wh[...], preferred_element_type=f32)
           + jnp.dot(sua[0] + sua[1], wu[...], preferred_element_type=f32)
           + jnp.dot(sus[0] + sus[1], wsh[...], preferred_element_type=f32)
           + b1[...][None, :])
    z = _leaky(pre)
    out[...] = jnp.tanh(jnp.dot(z, w2[...], preferred_element_type=f32)
                        + b2[...][None, :])


def _update(pos_state, h, sua, sus, upd_W1, upd_b1, upd_W2, upd_b2):
    f32 = jnp.float32
    grid = N_S // _BN
    return pl.pallas_call(
        _upd_body,
        out_shape=jax.ShapeDtypeStruct((N_S, H_DIM), f32),
        grid=(grid,),
        in_specs=[
            pl.BlockSpec((_BN, 2), lambda i: (i, 0)),
            pl.BlockSpec((_BN, H_DIM), lambda i: (i, 0)),
            pl.BlockSpec((NC, _BN, H_DIM), lambda i: (0, i, 0)),
            pl.BlockSpec((NC, _BN, H_DIM), lambda i: (0, i, 0)),
            pl.BlockSpec((2, MLP_H), lambda i: (0, 0)),
            pl.BlockSpec((H_DIM, MLP_H), lambda i: (0, 0)),
            pl.BlockSpec((H_DIM, MLP_H), lambda i: (0, 0)),
            pl.BlockSpec((H_DIM, MLP_H), lambda i: (0, 0)),
            pl.BlockSpec((MLP_H,), lambda i: (0,)),
            pl.BlockSpec((MLP_H, H_DIM), lambda i: (0, 0)),
            pl.BlockSpec((H_DIM,), lambda i: (0,)),
        ],
        out_specs=pl.BlockSpec((_BN, H_DIM), lambda i: (i, 0)),
    )(pos_state, h, sua, sus,
      upd_W1[0:2], upd_W1[2:2 + H_DIM], upd_W1[2 + H_DIM:2 + 2 * H_DIM],
      upd_W1[2 + 2 * H_DIM:], upd_b1, upd_W2, upd_b2)


def kernel(h, u, pos_state, pos_action, a2s_edge_index, a2s_dis,
           s2s_edge_index, s2s_dis,
           a2s_W1, a2s_b1, a2s_W2, a2s_b2,
           s2s_W1, s2s_b1, s2s_W2, s2s_b2,
           upd_W1, upd_b1, upd_W2, upd_b2):
    asrc = a2s_edge_index[0]
    adst = a2s_edge_index[1]
    ssrc = s2s_edge_index[0]
    sdst = s2s_edge_index[1]

    pa, qa, ps, qs = _projections(pos_action, u, pos_state, h,
                                  a2s_W1, a2s_b1, s2s_W1, s2s_b1)
    ga1, ga2, gs1, gs2 = _gather(pa, qa, ps, qs, asrc, adst, ssrc, sdst)

    wdis_a = a2s_W1[4:5]
    wdis_s = s2s_W1[4:5]
    msga = _messages(ga1, ga2, a2s_dis, wdis_a, a2s_W2, a2s_b2, E_A2S)
    msgs = _messages(gs1, gs2, s2s_dis, wdis_s, s2s_W2, s2s_b2, E_S2S)

    zeros = jnp.zeros((N_S, H_DIM), jnp.float32)
    sua, sus = _segment_sums(msga, adst, msgs, sdst, zeros)

    return _update(pos_state, h, sua, sus, upd_W1, upd_b1, upd_W2, upd_b2)


# speedup vs baseline: 5.0229x; 1.7858x over previous
import functools

import jax
import jax.numpy as jnp
from jax import lax
from jax.experimental import pallas as pl
from jax.experimental.pallas import tpu as pltpu
from jax.experimental.pallas import tpu_sc as plsc

N_S = 10000
N_A = 10000
E_A2S = 320000
E_S2S = 320000
H_DIM = 128
MLP_H = 32

NC = 2
NSUB = 16
NW = NC * NSUB
EPW = E_A2S // NW
CH = 40
NCHUNK = EPW // CH
NBUF = 4

ROWS_PT = 624
TAIL_BASE = ROWS_PT * NSUB
TAIL_ROWS = N_S - TAIL_BASE

_MESH = dict(core_axis_name="c", subcore_axis_name="s", num_cores=NC,
             num_subcores=NSUB)


def _leaky(x):
    return jnp.where(x >= 0, x, 0.01 * x)


def _proj_body(pos_action, u, pos_state, h,
               wa_pos, wa_u, ba1, wa_q,
               ws_pos, ws_h, bs1, ws_q,
               pa_o, qa_o, ps_o, qs_o):
    f32 = jnp.float32
    pa_o[...] = (jnp.dot(pos_action[...], wa_pos[...], preferred_element_type=f32)
                 + jnp.dot(u[...], wa_u[...], preferred_element_type=f32)
                 + ba1[...][None, :])
    qa_o[...] = jnp.dot(pos_state[...], wa_q[...], preferred_element_type=f32)
    ps_o[...] = (jnp.dot(pos_state[...], ws_pos[...], preferred_element_type=f32)
                 + jnp.dot(h[...], ws_h[...], preferred_element_type=f32)
                 + bs1[...][None, :])
    qs_o[...] = jnp.dot(pos_state[...], ws_q[...], preferred_element_type=f32)


def _projections(pos_action, u, pos_state, h, a2s_W1, a2s_b1, s2s_W1, s2s_b1):
    f32 = jnp.float32
    out_shape = [
        jax.ShapeDtypeStruct((N_A, MLP_H), f32),
        jax.ShapeDtypeStruct((N_S, MLP_H), f32),
        jax.ShapeDtypeStruct((N_S, MLP_H), f32),
        jax.ShapeDtypeStruct((N_S, MLP_H), f32),
    ]
    return pl.pallas_call(_proj_body, out_shape=out_shape)(
        pos_action, u, pos_state, h,
        a2s_W1[0:2], a2s_W1[5:], a2s_b1, a2s_W1[2:4],
        s2s_W1[0:2], s2s_W1[5:], s2s_b1, s2s_W1[2:4])


def _gather_body(pa, qa, ps, qs, asrc, adst, ssrc, sdst,
                 oa, os, sidx, didx, rows1, rows2,
                 sg0, sg1, sg2, sg3, sw0, sw1, sw2, sw3):
    wid = lax.axis_index("s") * NC + lax.axis_index("c")
    base0 = wid * EPW
    sg = (sg0, sg1, sg2, sg3)
    sw = (sw0, sw1, sw2, sw3)

    def add_rows(r):
        for e in range(CH):
            for hf in range(MLP_H // 16):
                sl = pl.ds(hf * 16, 16)
                rows1[r, e, sl] = rows1[r, e, sl] + rows2[r, e, sl]

    for tab1, tab2, src, dst, out in ((pa, qa, asrc, adst, oa),
                                      (ps, qs, ssrc, sdst, os)):
        pltpu.sync_copy(src.at[pl.ds(base0, EPW)], sidx)
        pltpu.sync_copy(dst.at[pl.ds(base0, EPW)], didx)

        def g_issue(i, r, tab1=tab1, tab2=tab2):
            s = pl.ds(i * CH, CH)
            pltpu.async_copy(tab1.at[sidx.at[s]], rows1.at[r], sg[r])
            pltpu.async_copy(tab2.at[didx.at[s]], rows2.at[r], sg[r])

        def g_wait(i, r, tab1=tab1, tab2=tab2):
            s = pl.ds(i * CH, CH)
            pltpu.make_async_copy(tab1.at[sidx.at[s]], rows1.at[r], sg[r]).wait()
            pltpu.make_async_copy(tab2.at[didx.at[s]], rows2.at[r], sg[r]).wait()

        def w_issue(i, r, out=out):
            pltpu.async_copy(rows1.at[r], out.at[pl.ds(base0 + i * CH, CH)],
                             sw[r])

        def w_wait(i, r, out=out):
            pltpu.make_async_copy(rows1.at[r],
                                  out.at[pl.ds(base0 + i * CH, CH)],
                                  sw[r]).wait()

        g_issue(0, 0)
        g_issue(1, 1)

        def step(g, _):
            for b in range(NBUF):
                i = g * NBUF + b
                rp = (b + 2) % NBUF
                g_wait(i, b)
                if b < 2:
                    @pl.when(g >= 1)
                    def _wb():
                        w_wait(i - 2, rp)
                else:
                    w_wait(i - 2, rp)
                add_rows(b)
                w_issue(i, b)
                g_issue(i + 2, rp)
            return _
        lax.fori_loop(0, (NCHUNK - 2) // NBUF, step, 0, unroll=False)

        for j, b in ((NCHUNK - 2, 0), (NCHUNK - 1, 1)):
            g_wait(j, b)
            w_wait(j - 2, (b + 2) % NBUF)
            add_rows(b)
            w_issue(j, b)
            w_wait(j, b)


def _gather(pa, qa, ps, qs, asrc, adst, ssrc, sdst):
    f32 = jnp.float32
    out_type = [
        jax.ShapeDtypeStruct((E_A2S, MLP_H), f32),
        jax.ShapeDtypeStruct((E_S2S, MLP_H), f32),
    ]
    fn = pl.kernel(
        _gather_body,
        out_type=out_type,
        mesh=plsc.VectorSubcoreMesh(**_MESH),
        compiler_params=pltpu.CompilerParams(use_tc_tiling_on_sc=False),
        scratch_types=[
            pltpu.VMEM((EPW,), jnp.int32),
            pltpu.VMEM((EPW,), jnp.int32),
            pltpu.VMEM((NBUF, CH, MLP_H), f32),
            pltpu.VMEM((NBUF, CH, MLP_H), f32),
        ] + [pltpu.SemaphoreType.DMA] * (2 * NBUF),
    )
    return fn(pa, qa, ps, qs, asrc, adst, ssrc, sdst)


_BE = 4000


def _msg_body(g1, dis, wdis, w2, b2, out):
    f32 = jnp.float32
    pre = g1[...] + jnp.dot(dis[...], wdis[...], preferred_element_type=f32)
    z = _leaky(pre)
    out[...] = jnp.tanh(jnp.dot(z, w2[...], preferred_element_type=f32)
                        + b2[...][None, :])


def _messages(g1, dis, wdis, w2, b2, n_edges):
    f32 = jnp.float32
    grid = n_edges // _BE
    return pl.pallas_call(
        _msg_body,
        out_shape=jax.ShapeDtypeStruct((n_edges, H_DIM), f32),
        grid=(grid,),
        in_specs=[
            pl.BlockSpec((_BE, MLP_H), lambda i: (i, 0)),
            pl.BlockSpec((_BE, 1), lambda i: (i, 0)),
            pl.BlockSpec((1, MLP_H), lambda i: (0, 0)),
            pl.BlockSpec((MLP_H, H_DIM), lambda i: (0, 0)),
            pl.BlockSpec((H_DIM,), lambda i: (0,)),
        ],
        out_specs=pl.BlockSpec((_BE, H_DIM), lambda i: (i, 0)),
    )(g1, dis, wdis, w2, b2)


def _scatter_body(msga, adst, msgs, sdst, zeros,
                  outa, outs, idx_v, rows_v, accum,
                  si0, si1, si2, si3, sm0, sm1, sm2, sm3,
                  ss0, ss1, ss2, ss3):
    cid = lax.axis_index("c")
    sid = lax.axis_index("s")
    wid = sid * NC + cid
    base0 = wid * EPW
    rbase = sid * ROWS_PT
    last = sid == NSUB - 1
    si = (si0, si1, si2, si3)
    sm = (sm0, sm1, sm2, sm3)
    ss = (ss0, ss1, ss2, ss3)
    for msg, dst, out in ((msga, adst, outa), (msgs, sdst, outs)):
        pltpu.sync_copy(zeros.at[pl.ds(rbase, ROWS_PT)],
                        accum.at[pl.ds(rbase, ROWS_PT)])

        @pl.when(last)
        def _zero_tail():
            pltpu.sync_copy(zeros.at[pl.ds(TAIL_BASE, TAIL_ROWS)],
                            accum.at[pl.ds(TAIL_BASE, TAIL_ROWS)])
        plsc.subcore_barrier()

        def in_issue(i, r, msg=msg, dst=dst):
            pltpu.async_copy(dst.at[pl.ds(base0 + i * CH, CH)],
                             idx_v.at[r], si[r])
            pltpu.async_copy(msg.at[pl.ds(base0 + i * CH, CH)],
                             rows_v.at[r], sm[r])

        def in_wait(i, r, msg=msg, dst=dst):
            pltpu.make_async_copy(dst.at[pl.ds(base0 + i * CH, CH)],
                                  idx_v.at[r], si[r]).wait()
            pltpu.make_async_copy(msg.at[pl.ds(base0 + i * CH, CH)],
                                  rows_v.at[r], sm[r]).wait()

        def sc_issue(r):
            pltpu.async_copy(rows_v.at[r], accum.at[idx_v.at[r]], ss[r],
                             add=True)

        def sc_wait(r):
            pltpu.make_async_copy(rows_v.at[r], accum.at[idx_v.at[r]],
                                  ss[r]).wait()

        in_issue(0, 0)
        in_issue(1, 1)

        def step(g, _):
            for b in range(NBUF):
                i = g * NBUF + b
                rp = (b + 2) % NBUF
                in_wait(i, b)
                if b < 2:
                    @pl.when(g >= 1)
                    def _sw():
                        sc_wait(rp)
                else:
                    sc_wait(rp)
                sc_issue(b)
                in_issue(i + 2, rp)
            return _
        lax.fori_loop(0, (NCHUNK - 2) // NBUF, step, 0, unroll=False)

        for j, b in ((NCHUNK - 2, 0), (NCHUNK - 1, 1)):
            in_wait(j, b)
            sc_wait((b + 2) % NBUF)
            sc_issue(b)
        sc_wait(0)
        sc_wait(1)
        plsc.subcore_barrier()
        pltpu.sync_copy(accum.at[pl.ds(rbase, ROWS_PT)],
                        out.at[cid, pl.ds(rbase, ROWS_PT)])

        @pl.when(last)
        def _dump_tail():
            pltpu.sync_copy(accum.at[pl.ds(TAIL_BASE, TAIL_ROWS)],
                            out.at[cid, pl.ds(TAIL_BASE, TAIL_ROWS)])
        plsc.subcore_barrier()


def _segment_sums(msga, adst, msgs, sdst, zeros):
    f32 = jnp.float32
    out_type = [
        jax.ShapeDtypeStruct((NC, N_S, H_DIM), f32),
        jax.ShapeDtypeStruct((NC, N_S, H_DIM), f32),
    ]
    fn = pl.kernel(
        _scatter_body,
        out_type=out_type,
        mesh=plsc.VectorSubcoreMesh(**_MESH),
        scratch_types=[
            pltpu.VMEM((NBUF, CH), jnp.int32),
            pltpu.VMEM((NBUF, CH, H_DIM), f32),
            pltpu.VMEM_SHARED((N_S, H_DIM), f32),
        ] + [pltpu.SemaphoreType.DMA] * (3 * NBUF),
    )
    return fn(msga, adst, msgs, sdst, zeros)


_BN = 2000


def _upd_body(pos_state, h, sua, sus, wp, wh, wu, wsh, b1, w2, b2, out):
    f32 = jnp.float32
    pre = (jnp.dot(pos_state[...], wp[...], preferred_element_type=f32)
           + jnp.dot(h[...], wh[...], preferred_element_type=f32)
           + jnp.dot(sua[0] + sua[1], wu[...], preferred_element_type=f32)
           + jnp.dot(sus[0] + sus[1], wsh[...], preferred_element_type=f32)
           + b1[...][None, :])
    z = _leaky(pre)
    out[...] = jnp.tanh(jnp.dot(z, w2[...], preferred_element_type=f32)
                        + b2[...][None, :])


def _update(pos_state, h, sua, sus, upd_W1, upd_b1, upd_W2, upd_b2):
    f32 = jnp.float32
    grid = N_S // _BN
    return pl.pallas_call(
        _upd_body,
        out_shape=jax.ShapeDtypeStruct((N_S, H_DIM), f32),
        grid=(grid,),
        in_specs=[
            pl.BlockSpec((_BN, 2), lambda i: (i, 0)),
            pl.BlockSpec((_BN, H_DIM), lambda i: (i, 0)),
            pl.BlockSpec((NC, _BN, H_DIM), lambda i: (0, i, 0)),
            pl.BlockSpec((NC, _BN, H_DIM), lambda i: (0, i, 0)),
            pl.BlockSpec((2, MLP_H), lambda i: (0, 0)),
            pl.BlockSpec((H_DIM, MLP_H), lambda i: (0, 0)),
            pl.BlockSpec((H_DIM, MLP_H), lambda i: (0, 0)),
            pl.BlockSpec((H_DIM, MLP_H), lambda i: (0, 0)),
            pl.BlockSpec((MLP_H,), lambda i: (0,)),
            pl.BlockSpec((MLP_H, H_DIM), lambda i: (0, 0)),
            pl.BlockSpec((H_DIM,), lambda i: (0,)),
        ],
        out_specs=pl.BlockSpec((_BN, H_DIM), lambda i: (i, 0)),
    )(pos_state, h, sua, sus,
      upd_W1[0:2], upd_W1[2:2 + H_DIM], upd_W1[2 + H_DIM:2 + 2 * H_DIM],
      upd_W1[2 + 2 * H_DIM:], upd_b1, upd_W2, upd_b2)


def kernel(h, u, pos_state, pos_action, a2s_edge_index, a2s_dis,
           s2s_edge_index, s2s_dis,
           a2s_W1, a2s_b1, a2s_W2, a2s_b2,
           s2s_W1, s2s_b1, s2s_W2, s2s_b2,
           upd_W1, upd_b1, upd_W2, upd_b2):
    asrc = a2s_edge_index[0]
    adst = a2s_edge_index[1]
    ssrc = s2s_edge_index[0]
    sdst = s2s_edge_index[1]

    pa, qa, ps, qs = _projections(pos_action, u, pos_state, h,
                                  a2s_W1, a2s_b1, s2s_W1, s2s_b1)
    ga, gs = _gather(pa, qa, ps, qs, asrc, adst, ssrc, sdst)

    wdis_a = a2s_W1[4:5]
    wdis_s = s2s_W1[4:5]
    msga = _messages(ga, a2s_dis, wdis_a, a2s_W2, a2s_b2, E_A2S)
    msgs = _messages(gs, s2s_dis, wdis_s, s2s_W2, s2s_b2, E_S2S)

    zeros = jnp.zeros((N_S, H_DIM), jnp.float32)
    sua, sus = _segment_sums(msga, adst, msgs, sdst, zeros)

    return _update(pos_state, h, sua, sus, upd_W1, upd_b1, upd_W2, upd_b2)


# speedup vs baseline: 5.4151x; 1.0781x over previous
import functools

import jax
import jax.numpy as jnp
from jax import lax
from jax.experimental import pallas as pl
from jax.experimental.pallas import tpu as pltpu
from jax.experimental.pallas import tpu_sc as plsc

N_S = 10000
N_A = 10000
E_A2S = 320000
E_S2S = 320000
H_DIM = 128
MLP_H = 32

NC = 2
NSUB = 16
NW = NC * NSUB
EPW = E_A2S // NW
CH = 40
NCHUNK = EPW // CH
NBUF = 4

ROWS_PT = 624
TAIL_BASE = ROWS_PT * NSUB
TAIL_ROWS = N_S - TAIL_BASE

_MESH = dict(core_axis_name="c", subcore_axis_name="s", num_cores=NC,
             num_subcores=NSUB)


def _leaky(x):
    return jnp.where(x >= 0, x, 0.01 * x)


def _proj_body(pos_action, u, pos_state, h,
               wa_pos, wa_u, ba1, wa_q,
               ws_pos, ws_h, bs1, ws_q,
               pa_o, qa_o, ps_o, qs_o):
    f32 = jnp.float32
    pa_o[...] = (jnp.dot(pos_action[...], wa_pos[...], preferred_element_type=f32)
                 + jnp.dot(u[...], wa_u[...], preferred_element_type=f32)
                 + ba1[...][None, :])
    qa_o[...] = jnp.dot(pos_state[...], wa_q[...], preferred_element_type=f32)
    ps_o[...] = (jnp.dot(pos_state[...], ws_pos[...], preferred_element_type=f32)
                 + jnp.dot(h[...], ws_h[...], preferred_element_type=f32)
                 + bs1[...][None, :])
    qs_o[...] = jnp.dot(pos_state[...], ws_q[...], preferred_element_type=f32)


def _projections(pos_action, u, pos_state, h, a2s_W1, a2s_b1, s2s_W1, s2s_b1):
    f32 = jnp.float32
    out_shape = [
        jax.ShapeDtypeStruct((N_A, MLP_H), f32),
        jax.ShapeDtypeStruct((N_S, MLP_H), f32),
        jax.ShapeDtypeStruct((N_S, MLP_H), f32),
        jax.ShapeDtypeStruct((N_S, MLP_H), f32),
    ]
    return pl.pallas_call(_proj_body, out_shape=out_shape)(
        pos_action, u, pos_state, h,
        a2s_W1[0:2], a2s_W1[5:], a2s_b1, a2s_W1[2:4],
        s2s_W1[0:2], s2s_W1[5:], s2s_b1, s2s_W1[2:4])


def _gather_body(pa, qa, ps, qs, asrc, adst, ssrc, sdst,
                 out2, sidx, didx, rows1, rows2,
                 sg0, sg1, sg2, sg3, sw0, sw1, sw2, sw3):
    wid = lax.axis_index("s") * NC + lax.axis_index("c")
    base0 = wid * EPW
    sg = (sg0, sg1, sg2, sg3)
    sw = (sw0, sw1, sw2, sw3)

    def add_rows(r):
        for e in range(CH):
            for hf in range(MLP_H // 16):
                sl = pl.ds(hf * 16, 16)
                rows1[r, e, sl] = rows1[r, e, sl] + rows2[r, e, sl]

    for rel, (tab1, tab2, src, dst) in enumerate(((pa, qa, asrc, adst),
                                                  (ps, qs, ssrc, sdst))):
        pltpu.sync_copy(src.at[pl.ds(base0, EPW)], sidx)
        pltpu.sync_copy(dst.at[pl.ds(base0, EPW)], didx)

        def g_issue(i, r, tab1=tab1, tab2=tab2):
            s = pl.ds(i * CH, CH)
            pltpu.async_copy(tab1.at[sidx.at[s]], rows1.at[r], sg[r])
            pltpu.async_copy(tab2.at[didx.at[s]], rows2.at[r], sg[r])

        def g_wait(i, r, tab1=tab1, tab2=tab2):
            s = pl.ds(i * CH, CH)
            pltpu.make_async_copy(tab1.at[sidx.at[s]], rows1.at[r], sg[r]).wait()
            pltpu.make_async_copy(tab2.at[didx.at[s]], rows2.at[r], sg[r]).wait()

        def w_issue(i, r, rel=rel):
            pltpu.async_copy(rows1.at[r],
                             out2.at[rel, pl.ds(base0 + i * CH, CH)], sw[r])

        def w_wait(i, r, rel=rel):
            pltpu.make_async_copy(rows1.at[r],
                                  out2.at[rel, pl.ds(base0 + i * CH, CH)],
                                  sw[r]).wait()

        g_issue(0, 0)
        g_issue(1, 1)

        def step(g, _):
            for b in range(NBUF):
                i = g * NBUF + b
                rp = (b + 2) % NBUF
                g_wait(i, b)
                if b < 2:
                    @pl.when(g >= 1)
                    def _wb():
                        w_wait(i - 2, rp)
                else:
                    w_wait(i - 2, rp)
                add_rows(b)
                w_issue(i, b)
                g_issue(i + 2, rp)
            return _
        lax.fori_loop(0, (NCHUNK - 2) // NBUF, step, 0, unroll=False)

        for j, b in ((NCHUNK - 2, 0), (NCHUNK - 1, 1)):
            g_wait(j, b)
            w_wait(j - 2, (b + 2) % NBUF)
            add_rows(b)
            w_issue(j, b)
            w_wait(j, b)


def _gather(pa, qa, ps, qs, asrc, adst, ssrc, sdst):
    f32 = jnp.float32
    out_type = jax.ShapeDtypeStruct((2, E_A2S, MLP_H), f32)
    fn = pl.kernel(
        _gather_body,
        out_type=out_type,
        mesh=plsc.VectorSubcoreMesh(**_MESH),
        compiler_params=pltpu.CompilerParams(use_tc_tiling_on_sc=False),
        scratch_types=[
            pltpu.VMEM((EPW,), jnp.int32),
            pltpu.VMEM((EPW,), jnp.int32),
            pltpu.VMEM((NBUF, CH, MLP_H), f32),
            pltpu.VMEM((NBUF, CH, MLP_H), f32),
        ] + [pltpu.SemaphoreType.DMA] * (2 * NBUF),
    )
    return fn(pa, qa, ps, qs, asrc, adst, ssrc, sdst)


_BE = 3200


def _msg_body(g1, dis, wdis, w2, b2, out):
    f32 = jnp.float32
    r = pl.program_id(0)
    d = jnp.where(r == 0, dis[0], dis[1])
    pre = g1[0] + d[:, None] * wdis[0, 0][None, :]
    z = _leaky(pre)
    out[0] = jnp.tanh(jnp.dot(z, w2[0], preferred_element_type=f32)
                      + b2[0])


def _messages(g2, dis2, wdis2, w22, b22):
    f32 = jnp.float32
    grid = (2, E_A2S // _BE)
    return pl.pallas_call(
        _msg_body,
        out_shape=jax.ShapeDtypeStruct((2, E_A2S, H_DIM), f32),
        grid=grid,
        in_specs=[
            pl.BlockSpec((1, _BE, MLP_H), lambda r, i: (r, i, 0)),
            pl.BlockSpec((2, _BE), lambda r, i: (0, i)),
            pl.BlockSpec((1, 1, MLP_H), lambda r, i: (r, 0, 0)),
            pl.BlockSpec((1, MLP_H, H_DIM), lambda r, i: (r, 0, 0)),
            pl.BlockSpec((1, 1, H_DIM), lambda r, i: (r, 0, 0)),
        ],
        out_specs=pl.BlockSpec((1, _BE, H_DIM), lambda r, i: (r, i, 0)),
    )(g2, dis2, wdis2, w22, b22)


def _scatter_body(msg2, adst, sdst, zeros,
                  outa, outs, idx_v, rows_v, accum,
                  si0, si1, si2, si3, sm0, sm1, sm2, sm3,
                  ss0, ss1, ss2, ss3):
    cid = lax.axis_index("c")
    sid = lax.axis_index("s")
    wid = sid * NC + cid
    base0 = wid * EPW
    rbase = sid * ROWS_PT
    last = sid == NSUB - 1
    si = (si0, si1, si2, si3)
    sm = (sm0, sm1, sm2, sm3)
    ss = (ss0, ss1, ss2, ss3)
    for rel, (dst, out) in enumerate(((adst, outa), (sdst, outs))):
        pltpu.sync_copy(zeros.at[pl.ds(rbase, ROWS_PT)],
                        accum.at[pl.ds(rbase, ROWS_PT)])

        @pl.when(last)
        def _zero_tail():
            pltpu.sync_copy(zeros.at[pl.ds(TAIL_BASE, TAIL_ROWS)],
                            accum.at[pl.ds(TAIL_BASE, TAIL_ROWS)])
        plsc.subcore_barrier()

        def in_issue(i, r, rel=rel, dst=dst):
            pltpu.async_copy(dst.at[pl.ds(base0 + i * CH, CH)],
                             idx_v.at[r], si[r])
            pltpu.async_copy(msg2.at[rel, pl.ds(base0 + i * CH, CH)],
                             rows_v.at[r], sm[r])

        def in_wait(i, r, rel=rel, dst=dst):
            pltpu.make_async_copy(dst.at[pl.ds(base0 + i * CH, CH)],
                                  idx_v.at[r], si[r]).wait()
            pltpu.make_async_copy(msg2.at[rel, pl.ds(base0 + i * CH, CH)],
                                  rows_v.at[r], sm[r]).wait()

        def sc_issue(r):
            pltpu.async_copy(rows_v.at[r], accum.at[idx_v.at[r]], ss[r],
                             add=True)

        def sc_wait(r):
            pltpu.make_async_copy(rows_v.at[r], accum.at[idx_v.at[r]],
                                  ss[r]).wait()

        in_issue(0, 0)
        in_issue(1, 1)

        def step(g, _):
            for b in range(NBUF):
                i = g * NBUF + b
                rp = (b + 2) % NBUF
                in_wait(i, b)
                if b < 2:
                    @pl.when(g >= 1)
                    def _sw():
                        sc_wait(rp)
                else:
                    sc_wait(rp)
                sc_issue(b)
                in_issue(i + 2, rp)
            return _
        lax.fori_loop(0, (NCHUNK - 2) // NBUF, step, 0, unroll=False)

        for j, b in ((NCHUNK - 2, 0), (NCHUNK - 1, 1)):
            in_wait(j, b)
            sc_wait((b + 2) % NBUF)
            sc_issue(b)
        sc_wait(0)
        sc_wait(1)
        plsc.subcore_barrier()
        pltpu.sync_copy(accum.at[pl.ds(rbase, ROWS_PT)],
                        out.at[cid, pl.ds(rbase, ROWS_PT)])

        @pl.when(last)
        def _dump_tail():
            pltpu.sync_copy(accum.at[pl.ds(TAIL_BASE, TAIL_ROWS)],
                            out.at[cid, pl.ds(TAIL_BASE, TAIL_ROWS)])
        plsc.subcore_barrier()


def _segment_sums(msg2, adst, sdst, zeros):
    f32 = jnp.float32
    out_type = [
        jax.ShapeDtypeStruct((NC, N_S, H_DIM), f32),
        jax.ShapeDtypeStruct((NC, N_S, H_DIM), f32),
    ]
    fn = pl.kernel(
        _scatter_body,
        out_type=out_type,
        mesh=plsc.VectorSubcoreMesh(**_MESH),
        scratch_types=[
            pltpu.VMEM((NBUF, CH), jnp.int32),
            pltpu.VMEM((NBUF, CH, H_DIM), f32),
            pltpu.VMEM_SHARED((N_S, H_DIM), f32),
        ] + [pltpu.SemaphoreType.DMA] * (3 * NBUF),
    )
    return fn(msg2, adst, sdst, zeros)


_BN = 2000


def _upd_body(pos_state, h, sua, sus, wp, wh, wu, wsh, b1, w2, b2, out):
    f32 = jnp.float32
    pre = (jnp.dot(pos_state[...], wp[...], preferred_element_type=f32)
           + jnp.dot(h[...], wh[...], preferred_element_type=f32)
           + jnp.dot(sua[0] + sua[1], wu[...], preferred_element_type=f32)
           + jnp.dot(sus[0] + sus[1], wsh[...], preferred_element_type=f32)
           + b1[...][None, :])
    z = _leaky(pre)
    out[...] = jnp.tanh(jnp.dot(z, w2[...], preferred_element_type=f32)
                        + b2[...][None, :])


def _update(pos_state, h, sua, sus, upd_W1, upd_b1, upd_W2, upd_b2):
    f32 = jnp.float32
    grid = N_S // _BN
    return pl.pallas_call(
        _upd_body,
        out_shape=jax.ShapeDtypeStruct((N_S, H_DIM), f32),
        grid=(grid,),
        in_specs=[
            pl.BlockSpec((_BN, 2), lambda i: (i, 0)),
            pl.BlockSpec((_BN, H_DIM), lambda i: (i, 0)),
            pl.BlockSpec((NC, _BN, H_DIM), lambda i: (0, i, 0)),
            pl.BlockSpec((NC, _BN, H_DIM), lambda i: (0, i, 0)),
            pl.BlockSpec((2, MLP_H), lambda i: (0, 0)),
            pl.BlockSpec((H_DIM, MLP_H), lambda i: (0, 0)),
            pl.BlockSpec((H_DIM, MLP_H), lambda i: (0, 0)),
            pl.BlockSpec((H_DIM, MLP_H), lambda i: (0, 0)),
            pl.BlockSpec((MLP_H,), lambda i: (0,)),
            pl.BlockSpec((MLP_H, H_DIM), lambda i: (0, 0)),
            pl.BlockSpec((H_DIM,), lambda i: (0,)),
        ],
        out_specs=pl.BlockSpec((_BN, H_DIM), lambda i: (i, 0)),
    )(pos_state, h, sua, sus,
      upd_W1[0:2], upd_W1[2:2 + H_DIM], upd_W1[2 + H_DIM:2 + 2 * H_DIM],
      upd_W1[2 + 2 * H_DIM:], upd_b1, upd_W2, upd_b2)


def kernel(h, u, pos_state, pos_action, a2s_edge_index, a2s_dis,
           s2s_edge_index, s2s_dis,
           a2s_W1, a2s_b1, a2s_W2, a2s_b2,
           s2s_W1, s2s_b1, s2s_W2, s2s_b2,
           upd_W1, upd_b1, upd_W2, upd_b2):
    asrc = a2s_edge_index[0]
    adst = a2s_edge_index[1]
    ssrc = s2s_edge_index[0]
    sdst = s2s_edge_index[1]

    pa, qa, ps, qs = _projections(pos_action, u, pos_state, h,
                                  a2s_W1, a2s_b1, s2s_W1, s2s_b1)
    g2 = _gather(pa, qa, ps, qs, asrc, adst, ssrc, sdst)

    dis2 = jnp.stack([a2s_dis[:, 0], s2s_dis[:, 0]])
    wdis2 = jnp.stack([a2s_W1[4:5], s2s_W1[4:5]])
    w22 = jnp.stack([a2s_W2, s2s_W2])
    b22 = jnp.stack([a2s_b2, s2s_b2])[:, None, :]
    msg2 = _messages(g2, dis2, wdis2, w22, b22)

    zeros = jnp.zeros((N_S, H_DIM), jnp.float32)
    sua, sus = _segment_sums(msg2, adst, sdst, zeros)

    return _update(pos_state, h, sua, sus, upd_W1, upd_b1, upd_W2, upd_b2)


# speedup vs baseline: 5.6232x; 1.0384x over previous
import functools

import jax
import jax.numpy as jnp
from jax import lax
from jax.experimental import pallas as pl
from jax.experimental.pallas import tpu as pltpu
from jax.experimental.pallas import tpu_sc as plsc

N_S = 10000
N_A = 10000
E_A2S = 320000
E_S2S = 320000
H_DIM = 128
MLP_H = 32

NC = 2
NSUB = 16
NW = NC * NSUB
EPW = E_A2S // NW
CH = 80
NCHUNK = EPW // CH
NBUF = 4
MAIN = (NCHUNK - 2) // NBUF

ROWS_PT = 624
TAIL_BASE = ROWS_PT * NSUB
TAIL_ROWS = N_S - TAIL_BASE

_MESH = dict(core_axis_name="c", subcore_axis_name="s", num_cores=NC,
             num_subcores=NSUB)


def _leaky(x):
    return jnp.where(x >= 0, x, 0.01 * x)


def _proj_body(pos_action, u, pos_state, h,
               wa_pos, wa_u, ba1, wa_q,
               ws_pos, ws_h, bs1, ws_q,
               pa_o, qa_o, ps_o, qs_o):
    f32 = jnp.float32
    pa_o[...] = (jnp.dot(pos_action[...], wa_pos[...], preferred_element_type=f32)
                 + jnp.dot(u[...], wa_u[...], preferred_element_type=f32)
                 + ba1[...][None, :])
    qa_o[...] = jnp.dot(pos_state[...], wa_q[...], preferred_element_type=f32)
    ps_o[...] = (jnp.dot(pos_state[...], ws_pos[...], preferred_element_type=f32)
                 + jnp.dot(h[...], ws_h[...], preferred_element_type=f32)
                 + bs1[...][None, :])
    qs_o[...] = jnp.dot(pos_state[...], ws_q[...], preferred_element_type=f32)


def _projections(pos_action, u, pos_state, h, a2s_W1, a2s_b1, s2s_W1, s2s_b1):
    f32 = jnp.float32
    out_shape = [
        jax.ShapeDtypeStruct((N_A, MLP_H), f32),
        jax.ShapeDtypeStruct((N_S, MLP_H), f32),
        jax.ShapeDtypeStruct((N_S, MLP_H), f32),
        jax.ShapeDtypeStruct((N_S, MLP_H), f32),
    ]
    return pl.pallas_call(_proj_body, out_shape=out_shape)(
        pos_action, u, pos_state, h,
        a2s_W1[0:2], a2s_W1[5:], a2s_b1, a2s_W1[2:4],
        s2s_W1[0:2], s2s_W1[5:], s2s_b1, s2s_W1[2:4])


def _gather_body(pa, qa, ps, qs, asrc, adst, ssrc, sdst,
                 out2, sidx, didx, rows1, rows2,
                 sg0, sg1, sg2, sg3, sw0, sw1, sw2, sw3):
    wid = lax.axis_index("s") * NC + lax.axis_index("c")
    base0 = wid * EPW
    sg = (sg0, sg1, sg2, sg3)
    sw = (sw0, sw1, sw2, sw3)

    def add_rows(r):
        for e in range(CH):
            for hf in range(MLP_H // 16):
                sl = pl.ds(hf * 16, 16)
                rows1[r, e, sl] = rows1[r, e, sl] + rows2[r, e, sl]

    def add_rows_compact(r):
        def body(it, c):
            for hf in range(MLP_H // 16):
                sl = pl.ds(hf * 16, 16)
                rows1[r, it, sl] = rows1[r, it, sl] + rows2[r, it, sl]
            return c
        lax.fori_loop(0, CH, body, 0, unroll=8)

    for rel, (tab1, tab2, src, dst) in enumerate(((pa, qa, asrc, adst),
                                                  (ps, qs, ssrc, sdst))):
        pltpu.sync_copy(src.at[pl.ds(base0, EPW)], sidx)
        pltpu.sync_copy(dst.at[pl.ds(base0, EPW)], didx)

        def g_issue(i, r, tab1=tab1, tab2=tab2):
            s = pl.ds(i * CH, CH)
            pltpu.async_copy(tab1.at[sidx.at[s]], rows1.at[r], sg[r])
            pltpu.async_copy(tab2.at[didx.at[s]], rows2.at[r], sg[r])

        def g_wait(i, r, tab1=tab1, tab2=tab2):
            s = pl.ds(i * CH, CH)
            pltpu.make_async_copy(tab1.at[sidx.at[s]], rows1.at[r], sg[r]).wait()
            pltpu.make_async_copy(tab2.at[didx.at[s]], rows2.at[r], sg[r]).wait()

        def w_issue(i, r, rel=rel):
            pltpu.async_copy(rows1.at[r],
                             out2.at[rel, pl.ds(base0 + i * CH, CH)], sw[r])

        def w_wait(i, r, rel=rel):
            pltpu.make_async_copy(rows1.at[r],
                                  out2.at[rel, pl.ds(base0 + i * CH, CH)],
                                  sw[r]).wait()

        g_issue(0, 0)
        g_issue(1, 1)

        def step(g, _):
            for b in range(NBUF):
                i = g * NBUF + b
                rp = (b + 2) % NBUF
                g_wait(i, b)
                if b < 2:
                    @pl.when(g >= 1)
                    def _wb():
                        w_wait(i - 2, rp)
                else:
                    w_wait(i - 2, rp)
                add_rows(b)
                w_issue(i, b)
                g_issue(i + 2, rp)
            return _
        lax.fori_loop(0, MAIN, step, 0, unroll=False)

        for j in range(MAIN * NBUF, NCHUNK):
            b = j % NBUF
            g_wait(j, b)
            w_wait(j - 2, (j - 2) % NBUF)
            add_rows_compact(b)
            w_issue(j, b)
            if j + 2 < NCHUNK:
                g_issue(j + 2, (j + 2) % NBUF)
        w_wait(NCHUNK - 2, (NCHUNK - 2) % NBUF)
        w_wait(NCHUNK - 1, (NCHUNK - 1) % NBUF)


def _gather(pa, qa, ps, qs, asrc, adst, ssrc, sdst):
    f32 = jnp.float32
    out_type = jax.ShapeDtypeStruct((2, E_A2S, MLP_H), f32)
    fn = pl.kernel(
        _gather_body,
        out_type=out_type,
        mesh=plsc.VectorSubcoreMesh(**_MESH),
        compiler_params=pltpu.CompilerParams(use_tc_tiling_on_sc=False),
        scratch_types=[
            pltpu.VMEM((EPW,), jnp.int32),
            pltpu.VMEM((EPW,), jnp.int32),
            pltpu.VMEM((NBUF, CH, MLP_H), f32),
            pltpu.VMEM((NBUF, CH, MLP_H), f32),
        ] + [pltpu.SemaphoreType.DMA] * (2 * NBUF),
    )
    return fn(pa, qa, ps, qs, asrc, adst, ssrc, sdst)


_BE = 3200


def _msg_body(g1, dis, wdis, w2, b2, out):
    f32 = jnp.float32
    r = pl.program_id(0)
    d = jnp.where(r == 0, dis[0], dis[1])
    pre = g1[0] + d[:, None] * wdis[0, 0][None, :]
    z = _leaky(pre)
    out[0] = jnp.tanh(jnp.dot(z, w2[0], preferred_element_type=f32)
                      + b2[0])


def _messages(g2, dis2, wdis2, w22, b22):
    f32 = jnp.float32
    grid = (2, E_A2S // _BE)
    return pl.pallas_call(
        _msg_body,
        out_shape=jax.ShapeDtypeStruct((2, E_A2S, H_DIM), f32),
        grid=grid,
        in_specs=[
            pl.BlockSpec((1, _BE, MLP_H), lambda r, i: (r, i, 0)),
            pl.BlockSpec((2, _BE), lambda r, i: (0, i)),
            pl.BlockSpec((1, 1, MLP_H), lambda r, i: (r, 0, 0)),
            pl.BlockSpec((1, MLP_H, H_DIM), lambda r, i: (r, 0, 0)),
            pl.BlockSpec((1, 1, H_DIM), lambda r, i: (r, 0, 0)),
        ],
        out_specs=pl.BlockSpec((1, _BE, H_DIM), lambda r, i: (r, i, 0)),
    )(g2, dis2, wdis2, w22, b22)


def _scatter_body(msg2, adst, sdst, zeros,
                  outa, outs, idx_v, rows_v, accum,
                  si0, si1, si2, si3, sm0, sm1, sm2, sm3,
                  ss0, ss1, ss2, ss3):
    cid = lax.axis_index("c")
    sid = lax.axis_index("s")
    wid = sid * NC + cid
    base0 = wid * EPW
    rbase = sid * ROWS_PT
    last = sid == NSUB - 1
    si = (si0, si1, si2, si3)
    sm = (sm0, sm1, sm2, sm3)
    ss = (ss0, ss1, ss2, ss3)
    for rel, (dst, out) in enumerate(((adst, outa), (sdst, outs))):
        pltpu.sync_copy(zeros.at[pl.ds(rbase, ROWS_PT)],
                        accum.at[pl.ds(rbase, ROWS_PT)])

        @pl.when(last)
        def _zero_tail():
            pltpu.sync_copy(zeros.at[pl.ds(TAIL_BASE, TAIL_ROWS)],
                            accum.at[pl.ds(TAIL_BASE, TAIL_ROWS)])
        plsc.subcore_barrier()

        def in_issue(i, r, rel=rel, dst=dst):
            pltpu.async_copy(dst.at[pl.ds(base0 + i * CH, CH)],
                             idx_v.at[r], si[r])
            pltpu.async_copy(msg2.at[rel, pl.ds(base0 + i * CH, CH)],
                             rows_v.at[r], sm[r])

        def in_wait(i, r, rel=rel, dst=dst):
            pltpu.make_async_copy(dst.at[pl.ds(base0 + i * CH, CH)],
                                  idx_v.at[r], si[r]).wait()
            pltpu.make_async_copy(msg2.at[rel, pl.ds(base0 + i * CH, CH)],
                                  rows_v.at[r], sm[r]).wait()

        def sc_issue(r):
            pltpu.async_copy(rows_v.at[r], accum.at[idx_v.at[r]], ss[r],
                             add=True)

        def sc_wait(r):
            pltpu.make_async_copy(rows_v.at[r], accum.at[idx_v.at[r]],
                                  ss[r]).wait()

        in_issue(0, 0)
        in_issue(1, 1)

        def step(g, _):
            for b in range(NBUF):
                i = g * NBUF + b
                rp = (b + 2) % NBUF
                in_wait(i, b)
                if b < 2:
                    @pl.when(g >= 1)
                    def _sw():
                        sc_wait(rp)
                else:
                    sc_wait(rp)
                sc_issue(b)
                in_issue(i + 2, rp)
            return _
        lax.fori_loop(0, MAIN, step, 0, unroll=False)

        for j in range(MAIN * NBUF, NCHUNK):
            b = j % NBUF
            in_wait(j, b)
            sc_wait((j - 2) % NBUF)
            sc_issue(b)
            if j + 2 < NCHUNK:
                in_issue(j + 2, (j + 2) % NBUF)
        sc_wait((NCHUNK - 2) % NBUF)
        sc_wait((NCHUNK - 1) % NBUF)
        plsc.subcore_barrier()
        pltpu.sync_copy(accum.at[pl.ds(rbase, ROWS_PT)],
                        out.at[cid, pl.ds(rbase, ROWS_PT)])

        @pl.when(last)
        def _dump_tail():
            pltpu.sync_copy(accum.at[pl.ds(TAIL_BASE, TAIL_ROWS)],
                            out.at[cid, pl.ds(TAIL_BASE, TAIL_ROWS)])
        plsc.subcore_barrier()


def _segment_sums(msg2, adst, sdst, zeros):
    f32 = jnp.float32
    out_type = [
        jax.ShapeDtypeStruct((NC, N_S, H_DIM), f32),
        jax.ShapeDtypeStruct((NC, N_S, H_DIM), f32),
    ]
    fn = pl.kernel(
        _scatter_body,
        out_type=out_type,
        mesh=plsc.VectorSubcoreMesh(**_MESH),
        scratch_types=[
            pltpu.VMEM((NBUF, CH), jnp.int32),
            pltpu.VMEM((NBUF, CH, H_DIM), f32),
            pltpu.VMEM_SHARED((N_S, H_DIM), f32),
        ] + [pltpu.SemaphoreType.DMA] * (3 * NBUF),
    )
    return fn(msg2, adst, sdst, zeros)


_BN = 2000


def _upd_body(pos_state, h, sua, sus, wp, wh, wu, wsh, b1, w2, b2, out):
    f32 = jnp.float32
    pre = (jnp.dot(pos_state[...], wp[...], preferred_element_type=f32)
           + jnp.dot(h[...], wh[...], preferred_element_type=f32)
           + jnp.dot(sua[0] + sua[1], wu[...], preferred_element_type=f32)
           + jnp.dot(sus[0] + sus[1], wsh[...], preferred_element_type=f32)
           + b1[...][None, :])
    z = _leaky(pre)
    out[...] = jnp.tanh(jnp.dot(z, w2[...], preferred_element_type=f32)
                        + b2[...][None, :])


def _update(pos_state, h, sua, sus, upd_W1, upd_b1, upd_W2, upd_b2):
    f32 = jnp.float32
    grid = N_S // _BN
    return pl.pallas_call(
        _upd_body,
        out_shape=jax.ShapeDtypeStruct((N_S, H_DIM), f32),
        grid=(grid,),
        in_specs=[
            pl.BlockSpec((_BN, 2), lambda i: (i, 0)),
            pl.BlockSpec((_BN, H_DIM), lambda i: (i, 0)),
            pl.BlockSpec((NC, _BN, H_DIM), lambda i: (0, i, 0)),
            pl.BlockSpec((NC, _BN, H_DIM), lambda i: (0, i, 0)),
            pl.BlockSpec((2, MLP_H), lambda i: (0, 0)),
            pl.BlockSpec((H_DIM, MLP_H), lambda i: (0, 0)),
            pl.BlockSpec((H_DIM, MLP_H), lambda i: (0, 0)),
            pl.BlockSpec((H_DIM, MLP_H), lambda i: (0, 0)),
            pl.BlockSpec((MLP_H,), lambda i: (0,)),
            pl.BlockSpec((MLP_H, H_DIM), lambda i: (0, 0)),
            pl.BlockSpec((H_DIM,), lambda i: (0,)),
        ],
        out_specs=pl.BlockSpec((_BN, H_DIM), lambda i: (i, 0)),
    )(pos_state, h, sua, sus,
      upd_W1[0:2], upd_W1[2:2 + H_DIM], upd_W1[2 + H_DIM:2 + 2 * H_DIM],
      upd_W1[2 + 2 * H_DIM:], upd_b1, upd_W2, upd_b2)


def kernel(h, u, pos_state, pos_action, a2s_edge_index, a2s_dis,
           s2s_edge_index, s2s_dis,
           a2s_W1, a2s_b1, a2s_W2, a2s_b2,
           s2s_W1, s2s_b1, s2s_W2, s2s_b2,
           upd_W1, upd_b1, upd_W2, upd_b2):
    asrc = a2s_edge_index[0]
    adst = a2s_edge_index[1]
    ssrc = s2s_edge_index[0]
    sdst = s2s_edge_index[1]

    pa, qa, ps, qs = _projections(pos_action, u, pos_state, h,
                                  a2s_W1, a2s_b1, s2s_W1, s2s_b1)
    g2 = _gather(pa, qa, ps, qs, asrc, adst, ssrc, sdst)

    dis2 = jnp.stack([a2s_dis[:, 0], s2s_dis[:, 0]])
    wdis2 = jnp.stack([a2s_W1[4:5], s2s_W1[4:5]])
    w22 = jnp.stack([a2s_W2, s2s_W2])
    b22 = jnp.stack([a2s_b2, s2s_b2])[:, None, :]
    msg2 = _messages(g2, dis2, wdis2, w22, b22)

    zeros = jnp.zeros((N_S, H_DIM), jnp.float32)
    sua, sus = _segment_sums(msg2, adst, sdst, zeros)

    return _update(pos_state, h, sua, sus, upd_W1, upd_b1, upd_W2, upd_b2)


# speedup vs baseline: 6.7618x; 1.2025x over previous
import functools

import jax
import jax.numpy as jnp
from jax import lax
from jax.experimental import pallas as pl
from jax.experimental.pallas import tpu as pltpu
from jax.experimental.pallas import tpu_sc as plsc

N_S = 10000
N_A = 10000
E_A2S = 320000
E_S2S = 320000
H_DIM = 128
MLP_H = 32

NC = 2
NSUB = 16
NW = NC * NSUB
EPW = E_A2S // NW
CH = 80
NCHUNK = EPW // CH
NBUF = 4
MAIN = (NCHUNK - 2) // NBUF

ROWS_PT = 624
TAIL_BASE = ROWS_PT * NSUB
TAIL_ROWS = N_S - TAIL_BASE

_MESH = dict(core_axis_name="c", subcore_axis_name="s", num_cores=NC,
             num_subcores=NSUB)


def _leaky(x):
    return jnp.where(x >= 0, x, 0.01 * x)


def _proj_body(pos_action, u, pos_state, h,
               wa_pos, wa_u, ba1, wa_q,
               ws_pos, ws_h, bs1, ws_q,
               pa_o, qa_o, ps_o, qs_o):
    f32 = jnp.float32
    pa_o[...] = (jnp.dot(pos_action[...], wa_pos[...], preferred_element_type=f32)
                 + jnp.dot(u[...], wa_u[...], preferred_element_type=f32)
                 + ba1[...][None, :])
    qa_o[...] = jnp.dot(pos_state[...], wa_q[...], preferred_element_type=f32)
    ps_o[...] = (jnp.dot(pos_state[...], ws_pos[...], preferred_element_type=f32)
                 + jnp.dot(h[...], ws_h[...], preferred_element_type=f32)
                 + bs1[...][None, :])
    qs_o[...] = jnp.dot(pos_state[...], ws_q[...], preferred_element_type=f32)


def _projections(pos_action, u, pos_state, h, a2s_W1, a2s_b1, s2s_W1, s2s_b1):
    f32 = jnp.float32
    out_shape = [
        jax.ShapeDtypeStruct((N_A, MLP_H), f32),
        jax.ShapeDtypeStruct((N_S, MLP_H), f32),
        jax.ShapeDtypeStruct((N_S, MLP_H), f32),
        jax.ShapeDtypeStruct((N_S, MLP_H), f32),
    ]
    return pl.pallas_call(_proj_body, out_shape=out_shape)(
        pos_action, u, pos_state, h,
        a2s_W1[0:2], a2s_W1[5:], a2s_b1, a2s_W1[2:4],
        s2s_W1[0:2], s2s_W1[5:], s2s_b1, s2s_W1[2:4])


def _gather_body(pa, qa, ps, qs, asrc, adst, ssrc, sdst,
                 out2, sidx, didx, rows1, rows2,
                 sg0, sg1, sg2, sg3, sw0, sw1, sw2, sw3):
    wid = lax.axis_index("s") * NC + lax.axis_index("c")
    base0 = wid * EPW
    sg = (sg0, sg1, sg2, sg3)
    sw = (sw0, sw1, sw2, sw3)
    def add_rows(r):
        for e in range(CH):
            for hf in range(MLP_H // 16):
                sl = pl.ds(hf * 16, 16)
                rows1[r, e, sl] = rows1[r, e, sl] + rows2[r, e, sl]

    def add_rows_compact(r):
        def body(j, c):
            for hf in range(MLP_H // 16):
                sl = pl.ds(hf * 16, 16)
                rows1[r, j, sl] = rows1[r, j, sl] + rows2[r, j, sl]
            return c
        lax.fori_loop(0, CH, body, 0, unroll=8)

    for rel, (tab1, tab2, src, dst) in enumerate(((pa, qa, asrc, adst),
                                                  (ps, qs, ssrc, sdst))):
        pltpu.sync_copy(src.at[pl.ds(base0, EPW)], sidx)
        pltpu.sync_copy(dst.at[pl.ds(base0, EPW)], didx)

        def g_issue(i, r, tab1=tab1, tab2=tab2):
            s = pl.ds(i * CH, CH)
            pltpu.async_copy(tab1.at[sidx.at[s]], rows1.at[r], sg[r])
            pltpu.async_copy(tab2.at[didx.at[s]], rows2.at[r], sg[r])

        def g_wait(i, r, tab1=tab1, tab2=tab2):
            s = pl.ds(i * CH, CH)
            pltpu.make_async_copy(tab1.at[sidx.at[s]], rows1.at[r], sg[r]).wait()
            pltpu.make_async_copy(tab2.at[didx.at[s]], rows2.at[r], sg[r]).wait()

        def _dst(i, rel):
            ge = base0 + i * CH
            rem = ge % _BE
            fb = (ge // _BE) * _BF + rem % _BF
            a0 = rem // _BF
            return out2.at[rel, pl.ds(fb, CH), pl.ds(a0 * MLP_H, MLP_H)]

        def w_issue(i, r, rel=rel):
            pltpu.async_copy(rows1.at[r], _dst(i, rel), sw[r])

        def w_wait(i, r, rel=rel):
            pltpu.make_async_copy(rows1.at[r], _dst(i, rel), sw[r]).wait()

        g_issue(0, 0)
        g_issue(1, 1)

        def step(g, _):
            for b in range(NBUF):
                i = g * NBUF + b
                rp = (b + 2) % NBUF
                g_wait(i, b)
                if b < 2:
                    @pl.when(g >= 1)
                    def _wb():
                        w_wait(i - 2, rp)
                else:
                    w_wait(i - 2, rp)
                add_rows(b)
                w_issue(i, b)
                g_issue(i + 2, rp)
            return _
        lax.fori_loop(0, MAIN, step, 0, unroll=False)

        for j in range(MAIN * NBUF, NCHUNK):
            b = j % NBUF
            g_wait(j, b)
            w_wait(j - 2, (j - 2) % NBUF)
            add_rows_compact(b)
            w_issue(j, b)
            if j + 2 < NCHUNK:
                g_issue(j + 2, (j + 2) % NBUF)
        w_wait(NCHUNK - 2, (NCHUNK - 2) % NBUF)
        w_wait(NCHUNK - 1, (NCHUNK - 1) % NBUF)


def _gather(pa, qa, ps, qs, asrc, adst, ssrc, sdst):
    f32 = jnp.float32
    out_type = jax.ShapeDtypeStruct((2, E_A2S // 4, 4 * MLP_H), f32)
    fn = pl.kernel(
        _gather_body,
        out_type=out_type,
        mesh=plsc.VectorSubcoreMesh(**_MESH),
        compiler_params=pltpu.CompilerParams(use_tc_tiling_on_sc=False),
        scratch_types=[
            pltpu.VMEM((EPW,), jnp.int32),
            pltpu.VMEM((EPW,), jnp.int32),
            pltpu.VMEM((NBUF, CH, MLP_H), f32),
            pltpu.VMEM((NBUF, CH, MLP_H), f32),
        ] + [pltpu.SemaphoreType.DMA] * (2 * NBUF),
    )
    return fn(pa, qa, ps, qs, asrc, adst, ssrc, sdst)


_BE = 3200
_BF = _BE // 4
_EF = E_A2S // 4


def _msg_body(gf, dis, wdis, w2, b2, out):
    f32 = jnp.float32
    r = pl.program_id(0)
    d = jnp.where(r == 0, dis[0], dis[1])
    w = wdis[0, 0]
    zf = gf[0]
    for a in range(4):
        d_a = d[a * _BF:(a + 1) * _BF]
        pre = zf[:, a * MLP_H:(a + 1) * MLP_H] + d_a[:, None] * w[None, :]
        z = _leaky(pre)
        out[0, pl.ds(a * _BF, _BF), :] = jnp.tanh(
            jnp.dot(z, w2[0], preferred_element_type=f32) + b2[0])


def _messages(g2, dis2, wdis2, w22, b22):
    f32 = jnp.float32
    grid = (2, E_A2S // _BE)
    return pl.pallas_call(
        _msg_body,
        out_shape=jax.ShapeDtypeStruct((2, E_A2S, H_DIM), f32),
        grid=grid,
        in_specs=[
            pl.BlockSpec((1, _BF, H_DIM), lambda r, i: (r, i, 0)),
            pl.BlockSpec((2, _BE), lambda r, i: (0, i)),
            pl.BlockSpec((1, 1, MLP_H), lambda r, i: (r, 0, 0)),
            pl.BlockSpec((1, MLP_H, H_DIM), lambda r, i: (r, 0, 0)),
            pl.BlockSpec((1, 1, H_DIM), lambda r, i: (r, 0, 0)),
        ],
        out_specs=pl.BlockSpec((1, _BE, H_DIM), lambda r, i: (r, i, 0)),
    )(g2, dis2, wdis2, w22, b22)


def _scatter_body(msg2, adst, sdst, zeros,
                  outa, outs, idx_v, rows_v, accum,
                  si0, si1, si2, si3, sm0, sm1, sm2, sm3,
                  ss0, ss1, ss2, ss3):
    cid = lax.axis_index("c")
    sid = lax.axis_index("s")
    wid = sid * NC + cid
    base0 = wid * EPW
    rbase = sid * ROWS_PT
    last = sid == NSUB - 1
    si = (si0, si1, si2, si3)
    sm = (sm0, sm1, sm2, sm3)
    ss = (ss0, ss1, ss2, ss3)
    for rel, (dst, out) in enumerate(((adst, outa), (sdst, outs))):
        pltpu.sync_copy(zeros.at[pl.ds(rbase, ROWS_PT)],
                        accum.at[pl.ds(rbase, ROWS_PT)])

        @pl.when(last)
        def _zero_tail():
            pltpu.sync_copy(zeros.at[pl.ds(TAIL_BASE, TAIL_ROWS)],
                            accum.at[pl.ds(TAIL_BASE, TAIL_ROWS)])
        plsc.subcore_barrier()

        def in_issue(i, r, rel=rel, dst=dst):
            pltpu.async_copy(dst.at[pl.ds(base0 + i * CH, CH)],
                             idx_v.at[r], si[r])
            pltpu.async_copy(msg2.at[rel, pl.ds(base0 + i * CH, CH)],
                             rows_v.at[r], sm[r])

        def in_wait(i, r, rel=rel, dst=dst):
            pltpu.make_async_copy(dst.at[pl.ds(base0 + i * CH, CH)],
                                  idx_v.at[r], si[r]).wait()
            pltpu.make_async_copy(msg2.at[rel, pl.ds(base0 + i * CH, CH)],
                                  rows_v.at[r], sm[r]).wait()

        def sc_issue(r):
            pltpu.async_copy(rows_v.at[r], accum.at[idx_v.at[r]], ss[r],
                             add=True)

        def sc_wait(r):
            pltpu.make_async_copy(rows_v.at[r], accum.at[idx_v.at[r]],
                                  ss[r]).wait()

        in_issue(0, 0)
        in_issue(1, 1)

        def step(g, _):
            for b in range(NBUF):
                i = g * NBUF + b
                rp = (b + 2) % NBUF
                in_wait(i, b)
                if b < 2:
                    @pl.when(g >= 1)
                    def _sw():
                        sc_wait(rp)
                else:
                    sc_wait(rp)
                sc_issue(b)
                in_issue(i + 2, rp)
            return _
        lax.fori_loop(0, MAIN, step, 0, unroll=False)

        for j in range(MAIN * NBUF, NCHUNK):
            b = j % NBUF
            in_wait(j, b)
            sc_wait((j - 2) % NBUF)
            sc_issue(b)
            if j + 2 < NCHUNK:
                in_issue(j + 2, (j + 2) % NBUF)
        sc_wait((NCHUNK - 2) % NBUF)
        sc_wait((NCHUNK - 1) % NBUF)
        plsc.subcore_barrier()
        pltpu.sync_copy(accum.at[pl.ds(rbase, ROWS_PT)],
                        out.at[cid, pl.ds(rbase, ROWS_PT)])

        @pl.when(last)
        def _dump_tail():
            pltpu.sync_copy(accum.at[pl.ds(TAIL_BASE, TAIL_ROWS)],
                            out.at[cid, pl.ds(TAIL_BASE, TAIL_ROWS)])
        plsc.subcore_barrier()


def _segment_sums(msg2, adst, sdst, zeros):
    f32 = jnp.float32
    out_type = [
        jax.ShapeDtypeStruct((NC, N_S, H_DIM), f32),
        jax.ShapeDtypeStruct((NC, N_S, H_DIM), f32),
    ]
    fn = pl.kernel(
        _scatter_body,
        out_type=out_type,
        mesh=plsc.VectorSubcoreMesh(**_MESH),
        scratch_types=[
            pltpu.VMEM((NBUF, CH), jnp.int32),
            pltpu.VMEM((NBUF, CH, H_DIM), f32),
            pltpu.VMEM_SHARED((N_S, H_DIM), f32),
        ] + [pltpu.SemaphoreType.DMA] * (3 * NBUF),
    )
    return fn(msg2, adst, sdst, zeros)


_BN = 2000


def _upd_body(pos_state, h, sua, sus, wp, wh, wu, wsh, b1, w2, b2, out):
    f32 = jnp.float32
    pre = (jnp.dot(pos_state[...], wp[...], preferred_element_type=f32)
           + jnp.dot(h[...], wh[...], preferred_element_type=f32)
           + jnp.dot(sua[0] + sua[1], wu[...], preferred_element_type=f32)
           + jnp.dot(sus[0] + sus[1], wsh[...], preferred_element_type=f32)
           + b1[...][None, :])
    z = _leaky(pre)
    out[...] = jnp.tanh(jnp.dot(z, w2[...], preferred_element_type=f32)
                        + b2[...][None, :])


def _update(pos_state, h, sua, sus, upd_W1, upd_b1, upd_W2, upd_b2):
    f32 = jnp.float32
    grid = N_S // _BN
    return pl.pallas_call(
        _upd_body,
        out_shape=jax.ShapeDtypeStruct((N_S, H_DIM), f32),
        grid=(grid,),
        in_specs=[
            pl.BlockSpec((_BN, 2), lambda i: (i, 0)),
            pl.BlockSpec((_BN, H_DIM), lambda i: (i, 0)),
            pl.BlockSpec((NC, _BN, H_DIM), lambda i: (0, i, 0)),
            pl.BlockSpec((NC, _BN, H_DIM), lambda i: (0, i, 0)),
            pl.BlockSpec((2, MLP_H), lambda i: (0, 0)),
            pl.BlockSpec((H_DIM, MLP_H), lambda i: (0, 0)),
            pl.BlockSpec((H_DIM, MLP_H), lambda i: (0, 0)),
            pl.BlockSpec((H_DIM, MLP_H), lambda i: (0, 0)),
            pl.BlockSpec((MLP_H,), lambda i: (0,)),
            pl.BlockSpec((MLP_H, H_DIM), lambda i: (0, 0)),
            pl.BlockSpec((H_DIM,), lambda i: (0,)),
        ],
        out_specs=pl.BlockSpec((_BN, H_DIM), lambda i: (i, 0)),
    )(pos_state, h, sua, sus,
      upd_W1[0:2], upd_W1[2:2 + H_DIM], upd_W1[2 + H_DIM:2 + 2 * H_DIM],
      upd_W1[2 + 2 * H_DIM:], upd_b1, upd_W2, upd_b2)


def kernel(h, u, pos_state, pos_action, a2s_edge_index, a2s_dis,
           s2s_edge_index, s2s_dis,
           a2s_W1, a2s_b1, a2s_W2, a2s_b2,
           s2s_W1, s2s_b1, s2s_W2, s2s_b2,
           upd_W1, upd_b1, upd_W2, upd_b2):
    asrc = a2s_edge_index[0]
    adst = a2s_edge_index[1]
    ssrc = s2s_edge_index[0]
    sdst = s2s_edge_index[1]

    pa, qa, ps, qs = _projections(pos_action, u, pos_state, h,
                                  a2s_W1, a2s_b1, s2s_W1, s2s_b1)
    g2f = _gather(pa, qa, ps, qs, asrc, adst, ssrc, sdst)

    dis2 = jnp.stack([a2s_dis[:, 0], s2s_dis[:, 0]])
    wdis2 = jnp.stack([a2s_W1[4:5], s2s_W1[4:5]])
    w22 = jnp.stack([a2s_W2, s2s_W2])
    b22 = jnp.stack([a2s_b2, s2s_b2])[:, None, :]
    msg2 = _messages(g2f, dis2, wdis2, w22, b22)

    zeros = jnp.zeros((N_S, H_DIM), jnp.float32)
    sua, sus = _segment_sums(msg2, adst, sdst, zeros)

    return _update(pos_state, h, sua, sus, upd_W1, upd_b1, upd_W2, upd_b2)
